# baseline (device time: 47017 ns/iter reference)
import jax
import jax.numpy as jnp
from jax import lax
from jax.experimental import pallas as pl
from jax.experimental.pallas import tpu as pltpu

N_DEV = 4
N_TOK = 512
D_IN = 256
D_OUT = 512
EXP_PER_DEV = 2
CAPACITY = 51


def kernel(x, router_W, route_idx, expert_W):
    del router_W

    def body(x_ref, idx_ref, w_ref, out_ref, comm_ref, send_sems, recv_sems):
        my_i = lax.axis_index("i")
        left = (my_i - 1) % N_DEV
        right = (my_i + 1) % N_DEV

        barrier_sem = pltpu.get_barrier_semaphore()
        for nbr in [left, right]:
            pl.semaphore_signal(
                barrier_sem, inc=1,
                device_id=(nbr,), device_id_type=pl.DeviceIdType.MESH,
            )
        pl.semaphore_wait(barrier_sem, 2)

        route = idx_ref[:, :]

        e_iota = lax.broadcasted_iota(jnp.int32, (N_TOK, EXP_PER_DEV), 1)
        is_e = (route == e_iota + EXP_PER_DEV * my_i).astype(jnp.float32)

        row = lax.broadcasted_iota(jnp.int32, (N_TOK, N_TOK), 0)
        col = lax.broadcasted_iota(jnp.int32, (N_TOK, N_TOK), 1)
        tri = (row > col).astype(jnp.float32)
        rank = jnp.dot(tri, is_e, preferred_element_type=jnp.float32)
        keep = is_e * (rank < CAPACITY).astype(jnp.float32)

        partial = keep[:, 0:1] * jnp.dot(
            x_ref[:, :], w_ref[0], preferred_element_type=jnp.float32
        ) + keep[:, 1:2] * jnp.dot(
            x_ref[:, :], w_ref[1], preferred_element_type=jnp.float32
        )

        out_ref[:, :] = partial
        comm_ref[0, :, :] = partial

        for h in range(N_DEV - 1):
            send_slot = h % 2
            recv_slot = (h + 1) % 2
            rdma = pltpu.make_async_remote_copy(
                src_ref=comm_ref.at[send_slot],
                dst_ref=comm_ref.at[recv_slot],
                send_sem=send_sems.at[send_slot],
                recv_sem=recv_sems.at[recv_slot],
                device_id=(right,),
                device_id_type=pl.DeviceIdType.MESH,
            )
            rdma.start()
            rdma.wait()
            out_ref[:, :] += comm_ref[recv_slot, :, :]

    return pl.pallas_call(
        body,
        out_shape=jax.ShapeDtypeStruct((N_TOK, D_OUT), jnp.float32),
        in_specs=[
            pl.BlockSpec(memory_space=pltpu.VMEM),
            pl.BlockSpec(memory_space=pltpu.VMEM),
            pl.BlockSpec(memory_space=pltpu.VMEM),
        ],
        out_specs=pl.BlockSpec(memory_space=pltpu.VMEM),
        scratch_shapes=[
            pltpu.VMEM((2, N_TOK, D_OUT), jnp.float32),
            pltpu.SemaphoreType.DMA((2,)),
            pltpu.SemaphoreType.DMA((2,)),
        ],
        compiler_params=pltpu.CompilerParams(collective_id=0),
    )(x, route_idx, expert_W)


# device time: 22021 ns/iter; 2.1351x vs baseline; 2.1351x over previous
import jax
import jax.numpy as jnp
from jax import lax
from jax.experimental import pallas as pl
from jax.experimental.pallas import tpu as pltpu

N_DEV = 4
N_TOK = 512
D_IN = 256
D_OUT = 512
EXP_PER_DEV = 2
CAPACITY = 51
SLOTS = 64
CHUNK = EXP_PER_DEV * SLOTS


def kernel(x, router_W, route_idx, expert_W):
    del router_W

    def body(x_ref, idx_ref, w_ref, out_ref, comm_ref, send_sems, recv_sems):
        my_i = lax.axis_index("i")
        left = (my_i - 1) % N_DEV
        right = (my_i + 1) % N_DEV

        barrier_sem = pltpu.get_barrier_semaphore()
        for nbr in [left, right]:
            pl.semaphore_signal(
                barrier_sem, inc=1,
                device_id=(nbr,), device_id_type=pl.DeviceIdType.MESH,
            )
        pl.semaphore_wait(barrier_sem, 2)

        route = idx_ref[:, :]

        e_iota = lax.broadcasted_iota(jnp.int32, (N_TOK, 8), 1)
        is_all = (route == e_iota).astype(jnp.float32)
        row = lax.broadcasted_iota(jnp.int32, (N_TOK, N_TOK), 0)
        col = lax.broadcasted_iota(jnp.int32, (N_TOK, N_TOK), 1)
        tri = (row > col).astype(jnp.float32)
        rank_all = jnp.dot(tri, is_all, preferred_element_type=jnp.float32)
        rank_own = jnp.sum(is_all * rank_all, axis=1, keepdims=True).astype(
            jnp.int32
        )
        in_cap = rank_own < CAPACITY

        s_iota = lax.broadcasted_iota(jnp.int32, (N_TOK, CHUNK), 1)

        def make_P(o):
            return (
                (route == EXP_PER_DEV * o + (s_iota >> 6))
                & (rank_own == (s_iota & (SLOTS - 1)))
                & in_cap
            ).astype(jnp.float32)

        P_me = make_P(my_i)
        xa = lax.dot_general(
            P_me, x_ref[:, :],
            dimension_numbers=(((0,), (0,)), ((), ())),
            preferred_element_type=jnp.float32,
        )
        comm_ref[0, 0:SLOTS, :] = jnp.dot(
            xa[0:SLOTS], w_ref[0], preferred_element_type=jnp.float32
        )
        comm_ref[0, SLOTS:CHUNK, :] = jnp.dot(
            xa[SLOTS:CHUNK], w_ref[1], preferred_element_type=jnp.float32
        )

        for h in range(N_DEV - 1):
            send_slot = h % 2
            recv_slot = (h + 1) % 2
            rdma = pltpu.make_async_remote_copy(
                src_ref=comm_ref.at[send_slot],
                dst_ref=comm_ref.at[recv_slot],
                send_sem=send_sems.at[send_slot],
                recv_sem=recv_sems.at[recv_slot],
                device_id=(right,),
                device_id_type=pl.DeviceIdType.MESH,
            )
            rdma.start()
            origin = (my_i - h) % N_DEV
            scatter = jnp.dot(
                make_P(origin), comm_ref[send_slot, :, :],
                preferred_element_type=jnp.float32,
            )
            if h == 0:
                out_ref[:, :] = scatter
            else:
                out_ref[:, :] += scatter
            rdma.wait()

        origin = (my_i - (N_DEV - 1)) % N_DEV
        out_ref[:, :] += jnp.dot(
            make_P(origin), comm_ref[(N_DEV - 1) % 2, :, :],
            preferred_element_type=jnp.float32,
        )

    return pl.pallas_call(
        body,
        out_shape=jax.ShapeDtypeStruct((N_TOK, D_OUT), jnp.float32),
        in_specs=[
            pl.BlockSpec(memory_space=pltpu.VMEM),
            pl.BlockSpec(memory_space=pltpu.VMEM),
            pl.BlockSpec(memory_space=pltpu.VMEM),
        ],
        out_specs=pl.BlockSpec(memory_space=pltpu.VMEM),
        scratch_shapes=[
            pltpu.VMEM((2, CHUNK, D_OUT), jnp.float32),
            pltpu.SemaphoreType.DMA((2,)),
            pltpu.SemaphoreType.DMA((2,)),
        ],
        compiler_params=pltpu.CompilerParams(collective_id=0),
    )(x, route_idx, expert_W)


# device time: 17173 ns/iter; 2.7378x vs baseline; 1.2823x over previous
import functools

import jax
import jax.numpy as jnp
from jax import lax
from jax.experimental import pallas as pl
from jax.experimental.pallas import tpu as pltpu

N_DEV = 4
N_TOK = 512
D_IN = 256
D_OUT = 512
EXP_PER_DEV = 2
CAPACITY = 51
SLOTS = 64
CHUNK = EXP_PER_DEV * SLOTS


def kernel(x, router_W, route_idx, expert_W):
    del router_W

    def body(x_ref, idx_ref, w_ref, out_ref, comm_ref, send_sems, recv_sems):
        my_i = lax.axis_index("i")
        left = (my_i - 1) % N_DEV
        right = (my_i + 1) % N_DEV
        diag = (my_i + 2) % N_DEV

        route = idx_ref[:, :]

        e_iota = lax.broadcasted_iota(jnp.int32, (N_TOK, 8), 1)
        is_all = (route == e_iota).astype(jnp.float32)
        row = lax.broadcasted_iota(jnp.int32, (N_TOK, N_TOK), 0)
        col = lax.broadcasted_iota(jnp.int32, (N_TOK, N_TOK), 1)
        tri = (row > col).astype(jnp.float32)
        rank_all = jnp.dot(tri, is_all, preferred_element_type=jnp.float32)
        rank_own = jnp.sum(is_all * rank_all, axis=1, keepdims=True).astype(
            jnp.int32
        )
        in_cap = rank_own < CAPACITY

        s_iota = lax.broadcasted_iota(jnp.int32, (N_TOK, CHUNK), 1)

        def make_P(o):
            return (
                (route == EXP_PER_DEV * o + (s_iota >> 6))
                & (rank_own == (s_iota & (SLOTS - 1)))
                & in_cap
            ).astype(jnp.float32)

        P_me = make_P(my_i)
        xa = lax.dot_general(
            P_me, x_ref[:, :],
            dimension_numbers=(((0,), (0,)), ((), ())),
            preferred_element_type=jnp.float32,
        )
        comm_ref[0, 0:SLOTS, :] = jnp.dot(
            xa[0:SLOTS], w_ref[0], preferred_element_type=jnp.float32
        )
        comm_ref[0, SLOTS:CHUNK, :] = jnp.dot(
            xa[SLOTS:CHUNK], w_ref[1], preferred_element_type=jnp.float32
        )

        barrier_sem = pltpu.get_barrier_semaphore()
        for nbr in [left, right]:
            pl.semaphore_signal(
                barrier_sem, inc=1,
                device_id=(nbr,), device_id_type=pl.DeviceIdType.MESH,
            )
        pl.semaphore_wait(barrier_sem, 2)

        rdmas = []
        for k, (target, dst_slot) in enumerate(
            [(right, 1), (left, 2), (diag, 3)]
        ):
            rdma = pltpu.make_async_remote_copy(
                src_ref=comm_ref.at[0],
                dst_ref=comm_ref.at[dst_slot],
                send_sem=send_sems.at[k],
                recv_sem=recv_sems.at[dst_slot - 1],
                device_id=(target,),
                device_id_type=pl.DeviceIdType.MESH,
            )
            rdma.start()
            rdmas.append(rdma)

        out_ref[:, :] = jnp.dot(
            P_me, comm_ref[0, :, :], preferred_element_type=jnp.float32
        )

        for origin, rdma_i, slot in [(left, 0, 1), (right, 1, 2), (diag, 2, 3)]:
            rdmas[rdma_i].wait_recv()
            out_ref[:, :] += jnp.dot(
                make_P(origin), comm_ref[slot, :, :],
                preferred_element_type=jnp.float32,
            )
        for rdma in rdmas:
            rdma.wait_send()

        @functools.partial(
            pl.run_scoped, second_barrier=pltpu.SemaphoreType.REGULAR
        )
        def _(second_barrier):
            for nbr in [left, right]:
                pl.semaphore_signal(
                    second_barrier, inc=1,
                    device_id=(nbr,), device_id_type=pl.DeviceIdType.MESH,
                )
            pl.semaphore_wait(second_barrier, 2)

    return pl.pallas_call(
        body,
        out_shape=jax.ShapeDtypeStruct((N_TOK, D_OUT), jnp.float32),
        in_specs=[
            pl.BlockSpec(memory_space=pltpu.VMEM),
            pl.BlockSpec(memory_space=pltpu.VMEM),
            pl.BlockSpec(memory_space=pltpu.VMEM),
        ],
        out_specs=pl.BlockSpec(memory_space=pltpu.VMEM),
        scratch_shapes=[
            pltpu.VMEM((4, CHUNK, D_OUT), jnp.float32),
            pltpu.SemaphoreType.DMA((3,)),
            pltpu.SemaphoreType.DMA((3,)),
        ],
        compiler_params=pltpu.CompilerParams(collective_id=0),
    )(x, route_idx, expert_W)


# device time: 14232 ns/iter; 3.3036x vs baseline; 1.2066x over previous
import functools

import jax
import jax.numpy as jnp
from jax import lax
from jax.experimental import pallas as pl
from jax.experimental.pallas import tpu as pltpu

N_DEV = 4
N_TOK = 512
D_IN = 256
D_OUT = 512
EXP_PER_DEV = 2
CAPACITY = 51
SLOTS = 64
CHUNK = EXP_PER_DEV * SLOTS


def kernel(x, router_W, route_idx, expert_W):
    del router_W

    def body(x_ref, idx_ref, w_ref, out_ref, comm_ref, send_sems, recv_sems):
        my_i = lax.axis_index("i")
        left = (my_i - 1) % N_DEV
        right = (my_i + 1) % N_DEV
        diag = (my_i + 2) % N_DEV

        route = idx_ref[:, :]

        e_iota = lax.broadcasted_iota(jnp.int32, (N_TOK, 8), 1)
        is_all = (route == e_iota).astype(jnp.float32)
        row = lax.broadcasted_iota(jnp.int32, (N_TOK, N_TOK), 0)
        col = lax.broadcasted_iota(jnp.int32, (N_TOK, N_TOK), 1)
        tri = (row > col).astype(jnp.float32)
        rank_all = jnp.dot(tri, is_all, preferred_element_type=jnp.float32)
        rank_own = jnp.sum(is_all * rank_all, axis=1, keepdims=True).astype(
            jnp.int32
        )
        in_cap = rank_own < CAPACITY

        s_iota = lax.broadcasted_iota(jnp.int32, (N_TOK, CHUNK), 1)

        def make_P(o):
            return (
                (route == EXP_PER_DEV * o + (s_iota >> 6))
                & (rank_own == (s_iota & (SLOTS - 1)))
                & in_cap
            ).astype(jnp.bfloat16)

        P_me = make_P(my_i)
        xa = lax.dot_general(
            P_me, x_ref[:, :].astype(jnp.bfloat16),
            dimension_numbers=(((0,), (0,)), ((), ())),
            preferred_element_type=jnp.float32,
        ).astype(jnp.bfloat16)
        wb = w_ref[:, :, :].astype(jnp.bfloat16)
        comm_ref[0, 0:SLOTS, :] = jnp.dot(
            xa[0:SLOTS], wb[0], preferred_element_type=jnp.float32
        ).astype(jnp.bfloat16)
        comm_ref[0, SLOTS:CHUNK, :] = jnp.dot(
            xa[SLOTS:CHUNK], wb[1], preferred_element_type=jnp.float32
        ).astype(jnp.bfloat16)

        barrier_sem = pltpu.get_barrier_semaphore()
        for nbr in [left, right]:
            pl.semaphore_signal(
                barrier_sem, inc=1,
                device_id=(nbr,), device_id_type=pl.DeviceIdType.MESH,
            )
        pl.semaphore_wait(barrier_sem, 2)

        rdmas = []
        for k, (target, dst_slot) in enumerate(
            [(right, 1), (left, 2), (diag, 3)]
        ):
            rdma = pltpu.make_async_remote_copy(
                src_ref=comm_ref.at[0],
                dst_ref=comm_ref.at[dst_slot],
                send_sem=send_sems.at[k],
                recv_sem=recv_sems.at[dst_slot - 1],
                device_id=(target,),
                device_id_type=pl.DeviceIdType.MESH,
            )
            rdma.start()
            rdmas.append(rdma)

        out_ref[:, :] = jnp.dot(
            P_me, comm_ref[0, :, :], preferred_element_type=jnp.float32
        )

        for origin, rdma_i, slot in [(left, 0, 1), (right, 1, 2), (diag, 2, 3)]:
            rdmas[rdma_i].wait_recv()
            out_ref[:, :] += jnp.dot(
                make_P(origin), comm_ref[slot, :, :],
                preferred_element_type=jnp.float32,
            )
        for rdma in rdmas:
            rdma.wait_send()

        @functools.partial(
            pl.run_scoped, second_barrier=pltpu.SemaphoreType.REGULAR
        )
        def _(second_barrier):
            for nbr in [left, right]:
                pl.semaphore_signal(
                    second_barrier, inc=1,
                    device_id=(nbr,), device_id_type=pl.DeviceIdType.MESH,
                )
            pl.semaphore_wait(second_barrier, 2)

    return pl.pallas_call(
        body,
        out_shape=jax.ShapeDtypeStruct((N_TOK, D_OUT), jnp.float32),
        in_specs=[
            pl.BlockSpec(memory_space=pltpu.VMEM),
            pl.BlockSpec(memory_space=pltpu.VMEM),
            pl.BlockSpec(memory_space=pltpu.VMEM),
        ],
        out_specs=pl.BlockSpec(memory_space=pltpu.VMEM),
        scratch_shapes=[
            pltpu.VMEM((4, CHUNK, D_OUT), jnp.bfloat16),
            pltpu.SemaphoreType.DMA((3,)),
            pltpu.SemaphoreType.DMA((3,)),
        ],
        compiler_params=pltpu.CompilerParams(collective_id=0),
    )(x, route_idx, expert_W)
